# baseline (speedup 1.0000x reference)
import functools

import jax
import jax.numpy as jnp
from jax import lax
from jax.experimental import pallas as pl
from jax.experimental.pallas import tpu as pltpu
from jax.experimental.pallas import tpu_sc as plsc

L = 16
NC, NS = 2, 16
NW = NC * NS
CH = 16


def _sqrt16(x):
    xs = jnp.maximum(x, jnp.float32(1e-30))
    i = lax.bitcast_convert_type(xs, jnp.int32)
    i = jnp.int32(0x5F3759DF) - lax.shift_right_logical(i, jnp.int32(1))
    y = lax.bitcast_convert_type(i, jnp.float32)
    xh = xs * jnp.float32(0.5)
    for _ in range(3):
        y = y * (jnp.float32(1.5) - xh * y * y)
    return xs * y


def kernel(heads, relations, tails, entity_emb, relation_emb):
    B = heads.shape[0]
    N, D = entity_emb.shape
    b_per_w = B // NW
    n_chunks = b_per_w // CH
    ent3 = entity_emb.reshape(N // 8, 8, D)
    relp = jnp.pad(relation_emb, ((0, 0), (0, 128 - D)))
    rel3 = relp.reshape(relp.shape[0] // 8, 8, 128)
    mesh = plsc.VectorSubcoreMesh(core_axis_name="c", subcore_axis_name="s")

    @functools.partial(
        pl.kernel,
        mesh=mesh,
        compiler_params=pltpu.CompilerParams(needs_layout_passes=False),
        out_type=jax.ShapeDtypeStruct((B,), jnp.float32),
        scratch_types=[
            pltpu.VMEM((b_per_w,), jnp.int32),
            pltpu.VMEM((b_per_w,), jnp.int32),
            pltpu.VMEM((b_per_w,), jnp.int32),
            pltpu.VMEM((2, D, CH), jnp.float32),
            pltpu.VMEM((2, D, CH), jnp.float32),
            pltpu.VMEM((2, 128, CH), jnp.float32),
            pltpu.VMEM((CH,), jnp.float32),
            pltpu.SemaphoreType.DMA((2,)),
            pltpu.SemaphoreType.DMA((2,)),
            pltpu.SemaphoreType.DMA((2,)),
        ],
    )
    def sc_kernel(heads_hbm, rels_hbm, tails_hbm, ent_hbm, rel_hbm, out_hbm,
                  hidx, tidx, ridx, hbuf, tbuf, rbuf, outc,
                  sem_h, sem_r, sem_t):
        wid = lax.axis_index("s") * NC + lax.axis_index("c")
        base = wid * b_per_w
        pltpu.sync_copy(heads_hbm.at[pl.ds(base, b_per_w)], hidx)
        pltpu.sync_copy(tails_hbm.at[pl.ds(base, b_per_w)], tidx)
        pltpu.sync_copy(rels_hbm.at[pl.ds(base, b_per_w)], ridx)

        def fire(off, slot):
            vh = hidx[pl.ds(off, CH)]
            vt = tidx[pl.ds(off, CH)]
            vr = ridx[pl.ds(off, CH)]
            for k in range(CH):
                pltpu.async_copy(
                    ent_hbm.at[lax.shift_right_logical(vh[k], 3), vh[k] & 7],
                    hbuf.at[slot, :, k], sem_h.at[slot])
                pltpu.async_copy(
                    ent_hbm.at[lax.shift_right_logical(vt[k], 3), vt[k] & 7],
                    tbuf.at[slot, :, k], sem_t.at[slot])
                pltpu.async_copy(
                    rel_hbm.at[lax.shift_right_logical(vr[k], 3), vr[k] & 7],
                    rbuf.at[slot, :, k], sem_r.at[slot])

        def wait(slot):
            for k in range(CH):
                pltpu.make_async_copy(ent_hbm.at[0, 0],
                                      hbuf.at[slot, :, k],
                                      sem_h.at[slot]).wait()
                pltpu.make_async_copy(ent_hbm.at[0, 0],
                                      tbuf.at[slot, :, k],
                                      sem_t.at[slot]).wait()
                pltpu.make_async_copy(rel_hbm.at[0, 0],
                                      rbuf.at[slot, :, k],
                                      sem_r.at[slot]).wait()

        def compute(off, slot):
            acc = None
            for j in range(D):
                d = hbuf[slot, j, :] + rbuf[slot, j, :] - tbuf[slot, j, :]
                sq = d * d
                acc = sq if acc is None else acc + sq
            outc[...] = _sqrt16(acc)
            pltpu.sync_copy(outc, out_hbm.at[pl.ds(base + off, CH)])

        fire(0, 0)

        @pl.loop(0, n_chunks // 2)
        def _pair(p):
            off0 = p * (2 * CH)

            fire(off0 + CH, 1)
            wait(0)
            compute(off0, 0)

            @pl.when(off0 + 2 * CH < b_per_w)
            def _():
                fire(off0 + 2 * CH, 0)

            wait(1)
            compute(off0 + CH, 1)

    return sc_kernel(heads, relations, tails, ent3, rel3)


# speedup vs baseline: 2.6637x; 2.6637x over previous
import functools

import jax
import jax.numpy as jnp
from jax import lax
from jax.experimental import pallas as pl
from jax.experimental.pallas import tpu as pltpu
from jax.experimental.pallas import tpu_sc as plsc

L = 16
NC, NS = 2, 16
NW = NC * NS
CH = 64


def _sqrt16(x):
    xs = jnp.maximum(x, jnp.float32(1e-30))
    i = lax.bitcast_convert_type(xs, jnp.int32)
    i = jnp.int32(0x5F3759DF) - lax.shift_right_logical(i, jnp.int32(1))
    y = lax.bitcast_convert_type(i, jnp.float32)
    xh = xs * jnp.float32(0.5)
    for _ in range(3):
        y = y * (jnp.float32(1.5) - xh * y * y)
    return xs * y


def kernel(heads, relations, tails, entity_emb, relation_emb):
    B = heads.shape[0]
    N, D = entity_emb.shape
    b_per_w = B // NW
    n_chunks = b_per_w // CH
    ent3 = entity_emb.reshape(N // 8, 8, D)
    relp = jnp.pad(relation_emb, ((0, 0), (0, 128 - D)))
    mesh = plsc.VectorSubcoreMesh(core_axis_name="c", subcore_axis_name="s")

    @functools.partial(
        pl.kernel,
        mesh=mesh,
        compiler_params=pltpu.CompilerParams(needs_layout_passes=False),
        out_type=jax.ShapeDtypeStruct((B,), jnp.float32),
        scratch_types=[
            pltpu.VMEM((b_per_w,), jnp.int32),
            pltpu.VMEM((b_per_w,), jnp.int32),
            pltpu.VMEM((b_per_w,), jnp.int32),
            pltpu.VMEM((2, CH, D), jnp.float32),
            pltpu.VMEM((2, CH, D), jnp.float32),
            pltpu.VMEM((2, CH, 128), jnp.float32),
            pltpu.VMEM((CH,), jnp.float32),
            pltpu.SemaphoreType.DMA((2,)),
            pltpu.SemaphoreType.DMA((2,)),
            pltpu.SemaphoreType.DMA((2,)),
        ],
    )
    def sc_kernel(heads_hbm, rels_hbm, tails_hbm, ent_hbm, rel_hbm, out_hbm,
                  hidx, tidx, ridx, hslab, tslab, rrow, outc,
                  sem_h, sem_r, sem_t):
        wid = lax.axis_index("s") * NC + lax.axis_index("c")
        base = wid * b_per_w
        pltpu.sync_copy(heads_hbm.at[pl.ds(base, b_per_w)], hidx)
        pltpu.sync_copy(tails_hbm.at[pl.ds(base, b_per_w)], tidx)
        pltpu.sync_copy(rels_hbm.at[pl.ds(base, b_per_w)], ridx)
        lane = lax.iota(jnp.int32, L)

        def fire(off, slot):
            pltpu.async_copy(
                rel_hbm.at[ridx.at[pl.ds(off, CH)]], rrow.at[slot],
                sem_r.at[slot])
            for g in range(CH // L):
                vh = hidx[pl.ds(off + g * L, L)]
                vt = tidx[pl.ds(off + g * L, L)]
                for k16 in range(L):
                    k = g * L + k16
                    pltpu.async_copy(
                        ent_hbm.at[lax.shift_right_logical(vh[k16], 3),
                                   vh[k16] & 7],
                        hslab.at[slot, k], sem_h.at[slot])
                    pltpu.async_copy(
                        ent_hbm.at[lax.shift_right_logical(vt[k16], 3),
                                   vt[k16] & 7],
                        tslab.at[slot, k], sem_t.at[slot])

        def wait(slot):
            pltpu.make_async_copy(ent_hbm.at[pl.ds(0, CH), 0],
                                  hslab.at[slot], sem_h.at[slot]).wait()
            pltpu.make_async_copy(ent_hbm.at[pl.ds(0, CH), 0],
                                  tslab.at[slot], sem_t.at[slot]).wait()
            pltpu.make_async_copy(rel_hbm.at[pl.ds(0, CH)],
                                  rrow.at[slot], sem_r.at[slot]).wait()

        def compute(off, slot):
            for g in range(CH // L):
                vec = jnp.zeros((L,), jnp.float32)
                for k16 in range(L):
                    k = g * L + k16
                    acc = None
                    for c in range(D // L):
                        sl = pl.ds(c * L, L)
                        d = (hslab[slot, k, sl] + rrow[slot, k, sl]
                             - tslab[slot, k, sl])
                        sq = d * d
                        acc = sq if acc is None else acc + sq
                    s = jnp.sum(acc)
                    vec = jnp.where(lane == k16, s, vec)
                outc[pl.ds(g * L, L)] = _sqrt16(vec)
            pltpu.sync_copy(outc, out_hbm.at[pl.ds(base + off, CH)])

        fire(0, 0)

        @pl.loop(0, n_chunks // 2)
        def _pair(p):
            off0 = p * (2 * CH)

            fire(off0 + CH, 1)
            wait(0)
            compute(off0, 0)

            @pl.when(off0 + 2 * CH < b_per_w)
            def _():
                fire(off0 + 2 * CH, 0)

            wait(1)
            compute(off0 + CH, 1)

    return sc_kernel(heads, relations, tails, ent3, relp)


# speedup vs baseline: 2.7192x; 1.0208x over previous
import functools

import jax
import jax.numpy as jnp
from jax import lax
from jax.experimental import pallas as pl
from jax.experimental.pallas import tpu as pltpu
from jax.experimental.pallas import tpu_sc as plsc

L = 16
NC, NS = 2, 16
NW = NC * NS
CH = 32


def _sqrt16(x):
    xs = jnp.maximum(x, jnp.float32(1e-30))
    i = lax.bitcast_convert_type(xs, jnp.int32)
    i = jnp.int32(0x5F3759DF) - lax.shift_right_logical(i, jnp.int32(1))
    y = lax.bitcast_convert_type(i, jnp.float32)
    xh = xs * jnp.float32(0.5)
    for _ in range(3):
        y = y * (jnp.float32(1.5) - xh * y * y)
    return xs * y


def kernel(heads, relations, tails, entity_emb, relation_emb):
    B = heads.shape[0]
    N, D = entity_emb.shape
    b_per_w = B // NW
    n_chunks = b_per_w // CH
    ent3 = entity_emb.reshape(N // 8, 8, D)
    relp = jnp.pad(relation_emb, ((0, 0), (0, 128 - D)))
    mesh = plsc.VectorSubcoreMesh(core_axis_name="c", subcore_axis_name="s")

    @functools.partial(
        pl.kernel,
        mesh=mesh,
        compiler_params=pltpu.CompilerParams(needs_layout_passes=False),
        out_type=jax.ShapeDtypeStruct((B,), jnp.float32),
        scratch_types=[
            pltpu.VMEM((b_per_w,), jnp.int32),
            pltpu.VMEM((b_per_w,), jnp.int32),
            pltpu.VMEM((b_per_w,), jnp.int32),
            pltpu.VMEM((2, CH, D), jnp.float32),
            pltpu.VMEM((2, CH, D), jnp.float32),
            pltpu.VMEM((2, CH, 128), jnp.float32),
            pltpu.VMEM((CH,), jnp.float32),
            pltpu.SemaphoreType.DMA((2,)),
            pltpu.SemaphoreType.DMA((2,)),
            pltpu.SemaphoreType.DMA((2,)),
        ],
    )
    def sc_kernel(heads_hbm, rels_hbm, tails_hbm, ent_hbm, rel_hbm, out_hbm,
                  hidx, tidx, ridx, hslab, tslab, rrow, outc,
                  sem_h, sem_r, sem_t):
        wid = lax.axis_index("s") * NC + lax.axis_index("c")
        base = wid * b_per_w
        pltpu.sync_copy(heads_hbm.at[pl.ds(base, b_per_w)], hidx)
        pltpu.sync_copy(tails_hbm.at[pl.ds(base, b_per_w)], tidx)
        pltpu.sync_copy(rels_hbm.at[pl.ds(base, b_per_w)], ridx)
        lane = lax.iota(jnp.int32, L)

        def fire(off, slot):
            pltpu.async_copy(
                rel_hbm.at[ridx.at[pl.ds(off, CH)]], rrow.at[slot],
                sem_r.at[slot])
            for g in range(CH // L):
                vh = hidx[pl.ds(off + g * L, L)]
                vt = tidx[pl.ds(off + g * L, L)]
                for k16 in range(L):
                    k = g * L + k16
                    pltpu.async_copy(
                        ent_hbm.at[lax.shift_right_logical(vh[k16], 3),
                                   vh[k16] & 7],
                        hslab.at[slot, k], sem_h.at[slot])
                    pltpu.async_copy(
                        ent_hbm.at[lax.shift_right_logical(vt[k16], 3),
                                   vt[k16] & 7],
                        tslab.at[slot, k], sem_t.at[slot])

        def wait(slot):
            pltpu.make_async_copy(ent_hbm.at[pl.ds(0, CH), 0],
                                  hslab.at[slot], sem_h.at[slot]).wait()
            pltpu.make_async_copy(ent_hbm.at[pl.ds(0, CH), 0],
                                  tslab.at[slot], sem_t.at[slot]).wait()
            pltpu.make_async_copy(rel_hbm.at[pl.ds(0, CH)],
                                  rrow.at[slot], sem_r.at[slot]).wait()

        def compute(off, slot):
            for g in range(CH // L):
                vec = jnp.zeros((L,), jnp.float32)
                for k16 in range(L):
                    k = g * L + k16
                    acc = None
                    for c in range(D // L):
                        sl = pl.ds(c * L, L)
                        d = (hslab[slot, k, sl] + rrow[slot, k, sl]
                             - tslab[slot, k, sl])
                        sq = d * d
                        acc = sq if acc is None else acc + sq
                    s = jnp.sum(acc)
                    vec = jnp.where(lane == k16, s, vec)
                outc[pl.ds(g * L, L)] = _sqrt16(vec)
            pltpu.sync_copy(outc, out_hbm.at[pl.ds(base + off, CH)])

        fire(0, 0)

        @pl.loop(0, n_chunks // 2)
        def _pair(p):
            off0 = p * (2 * CH)

            fire(off0 + CH, 1)
            wait(0)
            compute(off0, 0)

            @pl.when(off0 + 2 * CH < b_per_w)
            def _():
                fire(off0 + 2 * CH, 0)

            wait(1)
            compute(off0 + CH, 1)

    return sc_kernel(heads, relations, tails, ent3, relp)


# speedup vs baseline: 2.7575x; 1.0141x over previous
import functools

import jax
import jax.numpy as jnp
from jax import lax
from jax.experimental import pallas as pl
from jax.experimental.pallas import tpu as pltpu
from jax.experimental.pallas import tpu_sc as plsc

L = 16
NC, NS = 2, 16
NW = NC * NS
CH = 16


def _sqrt16(x):
    xs = jnp.maximum(x, jnp.float32(1e-30))
    i = lax.bitcast_convert_type(xs, jnp.int32)
    i = jnp.int32(0x5F3759DF) - lax.shift_right_logical(i, jnp.int32(1))
    y = lax.bitcast_convert_type(i, jnp.float32)
    xh = xs * jnp.float32(0.5)
    for _ in range(3):
        y = y * (jnp.float32(1.5) - xh * y * y)
    return xs * y


def kernel(heads, relations, tails, entity_emb, relation_emb):
    B = heads.shape[0]
    N, D = entity_emb.shape
    b_per_w = B // NW
    n_chunks = b_per_w // CH
    ent3 = entity_emb.reshape(N // 8, 8, D)
    relp = jnp.pad(relation_emb, ((0, 0), (0, 128 - D)))
    mesh = plsc.VectorSubcoreMesh(core_axis_name="c", subcore_axis_name="s")

    @functools.partial(
        pl.kernel,
        mesh=mesh,
        compiler_params=pltpu.CompilerParams(needs_layout_passes=False),
        out_type=jax.ShapeDtypeStruct((B,), jnp.float32),
        scratch_types=[
            pltpu.VMEM((b_per_w,), jnp.int32),
            pltpu.VMEM((b_per_w,), jnp.int32),
            pltpu.VMEM((b_per_w,), jnp.int32),
            pltpu.VMEM((4, CH, D), jnp.float32),
            pltpu.VMEM((4, CH, D), jnp.float32),
            pltpu.VMEM((4, CH, 128), jnp.float32),
            pltpu.VMEM((CH,), jnp.float32),
            pltpu.SemaphoreType.DMA((4,)),
            pltpu.SemaphoreType.DMA((4,)),
            pltpu.SemaphoreType.DMA((4,)),
        ],
    )
    def sc_kernel(heads_hbm, rels_hbm, tails_hbm, ent_hbm, rel_hbm, out_hbm,
                  hidx, tidx, ridx, hslab, tslab, rrow, outc,
                  sem_h, sem_r, sem_t):
        wid = lax.axis_index("s") * NC + lax.axis_index("c")
        base = wid * b_per_w
        pltpu.sync_copy(heads_hbm.at[pl.ds(base, b_per_w)], hidx)
        pltpu.sync_copy(tails_hbm.at[pl.ds(base, b_per_w)], tidx)
        pltpu.sync_copy(rels_hbm.at[pl.ds(base, b_per_w)], ridx)
        lane = lax.iota(jnp.int32, L)

        def fire(off, slot):
            pltpu.async_copy(
                rel_hbm.at[ridx.at[pl.ds(off, CH)]], rrow.at[slot],
                sem_r.at[slot])
            vh = hidx[pl.ds(off, CH)]
            vt = tidx[pl.ds(off, CH)]
            for k in range(CH):
                pltpu.async_copy(
                    ent_hbm.at[lax.shift_right_logical(vh[k], 3), vh[k] & 7],
                    hslab.at[slot, k], sem_h.at[slot])
                pltpu.async_copy(
                    ent_hbm.at[lax.shift_right_logical(vt[k], 3), vt[k] & 7],
                    tslab.at[slot, k], sem_t.at[slot])

        def wait(slot):
            pltpu.make_async_copy(ent_hbm.at[pl.ds(0, CH), 0],
                                  hslab.at[slot], sem_h.at[slot]).wait()
            pltpu.make_async_copy(ent_hbm.at[pl.ds(0, CH), 0],
                                  tslab.at[slot], sem_t.at[slot]).wait()
            pltpu.make_async_copy(rel_hbm.at[pl.ds(0, CH)],
                                  rrow.at[slot], sem_r.at[slot]).wait()

        def compute(off, slot):
            vec = jnp.zeros((L,), jnp.float32)
            for k in range(CH):
                acc = None
                for c in range(D // L):
                    sl = pl.ds(c * L, L)
                    d = (hslab[slot, k, sl] + rrow[slot, k, sl]
                         - tslab[slot, k, sl])
                    sq = d * d
                    acc = sq if acc is None else acc + sq
                s = jnp.sum(acc)
                vec = jnp.where(lane == k, s, vec)
            outc[...] = _sqrt16(vec)
            pltpu.sync_copy(outc, out_hbm.at[pl.ds(base + off, CH)])

        fire(0, 0)
        fire(CH, 1)
        fire(2 * CH, 2)

        @pl.loop(0, n_chunks // 4)
        def _quad(q):
            off0 = q * (4 * CH)
            for b in range(4):
                nxt = off0 + (b + 3) * CH

                @pl.when(nxt < b_per_w)
                def _():
                    fire(nxt, (b + 3) % 4)

                wait(b)
                compute(off0 + b * CH, b)

    return sc_kernel(heads, relations, tails, ent3, relp)


# speedup vs baseline: 2.7949x; 1.0136x over previous
import functools

import jax
import jax.numpy as jnp
from jax import lax
from jax.experimental import pallas as pl
from jax.experimental.pallas import tpu as pltpu
from jax.experimental.pallas import tpu_sc as plsc

L = 16
NC, NS = 2, 16
NW = NC * NS
CH = 16


def _sqrt16(x):
    xs = jnp.maximum(x, jnp.float32(1e-30))
    i = lax.bitcast_convert_type(xs, jnp.int32)
    i = jnp.int32(0x5F3759DF) - lax.shift_right_logical(i, jnp.int32(1))
    y = lax.bitcast_convert_type(i, jnp.float32)
    xh = xs * jnp.float32(0.5)
    for _ in range(3):
        y = y * (jnp.float32(1.5) - xh * y * y)
    return xs * y


def kernel(heads, relations, tails, entity_emb, relation_emb):
    B = heads.shape[0]
    N, D = entity_emb.shape
    b_per_w = B // NW
    n_chunks = b_per_w // CH
    ent3 = entity_emb.reshape(N // 8, 8, D)
    relp = jnp.pad(relation_emb, ((0, 0), (0, 128 - D)))
    mesh = plsc.VectorSubcoreMesh(core_axis_name="c", subcore_axis_name="s")

    @functools.partial(
        pl.kernel,
        mesh=mesh,
        compiler_params=pltpu.CompilerParams(needs_layout_passes=False),
        out_type=jax.ShapeDtypeStruct((B,), jnp.float32),
        scratch_types=[
            pltpu.VMEM((b_per_w,), jnp.int32),
            pltpu.VMEM((b_per_w,), jnp.int32),
            pltpu.VMEM((b_per_w,), jnp.int32),
            pltpu.VMEM((2, CH, D), jnp.float32),
            pltpu.VMEM((2, CH, D), jnp.float32),
            pltpu.VMEM((2, CH, 128), jnp.float32),
            pltpu.VMEM((CH,), jnp.float32),
            pltpu.SemaphoreType.DMA((2,)),
            pltpu.SemaphoreType.DMA((2,)),
            pltpu.SemaphoreType.DMA((2,)),
        ],
    )
    def sc_kernel(heads_hbm, rels_hbm, tails_hbm, ent_hbm, rel_hbm, out_hbm,
                  hidx, tidx, ridx, hslab, tslab, rrow, outc,
                  sem_h, sem_r, sem_t):
        wid = lax.axis_index("s") * NC + lax.axis_index("c")
        base = wid * b_per_w
        pltpu.sync_copy(heads_hbm.at[pl.ds(base, b_per_w)], hidx)
        pltpu.sync_copy(tails_hbm.at[pl.ds(base, b_per_w)], tidx)
        pltpu.sync_copy(rels_hbm.at[pl.ds(base, b_per_w)], ridx)
        lane = lax.iota(jnp.int32, L)

        def fire(off, slot):
            pltpu.async_copy(
                rel_hbm.at[ridx.at[pl.ds(off, CH)]], rrow.at[slot],
                sem_r.at[slot])
            vh = hidx[pl.ds(off, CH)]
            vt = tidx[pl.ds(off, CH)]
            for k in range(CH):
                pltpu.async_copy(
                    ent_hbm.at[lax.shift_right_logical(vh[k], 3), vh[k] & 7],
                    hslab.at[slot, k], sem_h.at[slot])
                pltpu.async_copy(
                    ent_hbm.at[lax.shift_right_logical(vt[k], 3), vt[k] & 7],
                    tslab.at[slot, k], sem_t.at[slot])

        def wait(slot):
            pltpu.make_async_copy(ent_hbm.at[pl.ds(0, CH), 0],
                                  hslab.at[slot], sem_h.at[slot]).wait()
            pltpu.make_async_copy(ent_hbm.at[pl.ds(0, CH), 0],
                                  tslab.at[slot], sem_t.at[slot]).wait()
            pltpu.make_async_copy(rel_hbm.at[pl.ds(0, CH)],
                                  rrow.at[slot], sem_r.at[slot]).wait()

        perms = [(lane + sh) & (L - 1) for sh in (8, 4, 2, 1)]

        dnums = lax.GatherDimensionNumbers(
            offset_dims=(), collapsed_slice_dims=(0,), start_index_map=(0,))

        def shuffle(v, p):
            return lax.gather(v, p[:, None], dnums, slice_sizes=(1,),
                              mode=lax.GatherScatterMode.PROMISE_IN_BOUNDS)

        def tsum(v):
            for p in perms:
                v = v + shuffle(v, p)
            return v

        def compute(off, slot):
            vec = jnp.zeros((L,), jnp.float32)
            for k in range(CH):
                acc = None
                for c in range(D // L):
                    sl = pl.ds(c * L, L)
                    d = (hslab[slot, k, sl] + rrow[slot, k, sl]
                         - tslab[slot, k, sl])
                    sq = d * d
                    acc = sq if acc is None else acc + sq
                vec = jnp.where(lane == k, tsum(acc), vec)
            outc[...] = _sqrt16(vec)
            pltpu.sync_copy(outc, out_hbm.at[pl.ds(base + off, CH)])

        fire(0, 0)

        @pl.loop(0, n_chunks // 2)
        def _pair(p):
            off0 = p * (2 * CH)

            fire(off0 + CH, 1)
            wait(0)
            compute(off0, 0)

            @pl.when(off0 + 2 * CH < b_per_w)
            def _():
                fire(off0 + 2 * CH, 0)

            wait(1)
            compute(off0 + CH, 1)

    return sc_kernel(heads, relations, tails, ent3, relp)


# speedup vs baseline: 2.8007x; 1.0021x over previous
import functools

import jax
import jax.numpy as jnp
from jax import lax
from jax.experimental import pallas as pl
from jax.experimental.pallas import tpu as pltpu
from jax.experimental.pallas import tpu_sc as plsc

L = 16
NC, NS = 2, 16
NW = NC * NS
CH = 16


def _sqrt16(x):
    xs = jnp.maximum(x, jnp.float32(1e-30))
    i = lax.bitcast_convert_type(xs, jnp.int32)
    i = jnp.int32(0x5F3759DF) - lax.shift_right_logical(i, jnp.int32(1))
    y = lax.bitcast_convert_type(i, jnp.float32)
    xh = xs * jnp.float32(0.5)
    for _ in range(3):
        y = y * (jnp.float32(1.5) - xh * y * y)
    return xs * y


def kernel(heads, relations, tails, entity_emb, relation_emb):
    B = heads.shape[0]
    N, D = entity_emb.shape
    b_per_w = B // NW
    n_chunks = b_per_w // CH
    ent3 = entity_emb.reshape(N // 8, 8, D)
    relp = jnp.pad(relation_emb, ((0, 0), (0, 128 - D)))
    mesh = plsc.VectorSubcoreMesh(core_axis_name="c", subcore_axis_name="s")

    @functools.partial(
        pl.kernel,
        mesh=mesh,
        compiler_params=pltpu.CompilerParams(needs_layout_passes=False),
        out_type=jax.ShapeDtypeStruct((B,), jnp.float32),
        scratch_types=[
            pltpu.VMEM((b_per_w,), jnp.int32),
            pltpu.VMEM((b_per_w,), jnp.int32),
            pltpu.VMEM((b_per_w,), jnp.int32),
            pltpu.VMEM((2, CH, D), jnp.float32),
            pltpu.VMEM((2, CH, D), jnp.float32),
            pltpu.VMEM((2, CH, 128), jnp.float32),
            pltpu.VMEM((CH,), jnp.float32),
            pltpu.SemaphoreType.DMA((2,)),
            pltpu.SemaphoreType.DMA((2,)),
            pltpu.SemaphoreType.DMA((2,)),
        ],
    )
    def sc_kernel(heads_hbm, rels_hbm, tails_hbm, ent_hbm, rel_hbm, out_hbm,
                  hidx, tidx, ridx, hslab, tslab, rrow, outc,
                  sem_h, sem_r, sem_t):
        wid = lax.axis_index("s") * NC + lax.axis_index("c")
        base = wid * b_per_w
        pltpu.sync_copy(heads_hbm.at[pl.ds(base, b_per_w)], hidx)
        pltpu.sync_copy(tails_hbm.at[pl.ds(base, b_per_w)], tidx)
        pltpu.sync_copy(rels_hbm.at[pl.ds(base, b_per_w)], ridx)
        lane = lax.iota(jnp.int32, L)

        def fire(off, slot):
            pltpu.async_copy(
                rel_hbm.at[ridx.at[pl.ds(off, CH)]], rrow.at[slot],
                sem_r.at[slot])
            vh = hidx[pl.ds(off, CH)]
            vt = tidx[pl.ds(off, CH)]
            for k in range(CH):
                pltpu.async_copy(
                    ent_hbm.at[lax.shift_right_logical(vh[k], 3), vh[k] & 7],
                    hslab.at[slot, k], sem_h.at[slot])
                pltpu.async_copy(
                    ent_hbm.at[lax.shift_right_logical(vt[k], 3), vt[k] & 7],
                    tslab.at[slot, k], sem_t.at[slot])

        def wait(slot):
            pltpu.make_async_copy(ent_hbm.at[pl.ds(0, CH), 0],
                                  hslab.at[slot], sem_h.at[slot]).wait()
            pltpu.make_async_copy(ent_hbm.at[pl.ds(0, CH), 0],
                                  tslab.at[slot], sem_t.at[slot]).wait()
            pltpu.make_async_copy(rel_hbm.at[pl.ds(0, CH)],
                                  rrow.at[slot], sem_r.at[slot]).wait()

        def compute(off, slot):
            vec = jnp.zeros((L,), jnp.float32)
            for k in range(CH):
                acc = None
                for c in range(D // L):
                    sl = pl.ds(c * L, L)
                    d = (hslab[slot, k, sl] + rrow[slot, k, sl]
                         - tslab[slot, k, sl])
                    sq = d * d
                    acc = sq if acc is None else acc + sq
                s = jnp.sum(acc)
                vec = jnp.where(lane == k, s, vec)
            outc[...] = _sqrt16(vec)
            pltpu.sync_copy(outc, out_hbm.at[pl.ds(base + off, CH)])

        fire(0, 0)

        @pl.loop(0, n_chunks // 2)
        def _pair(p):
            off0 = p * (2 * CH)

            fire(off0 + CH, 1)
            wait(0)
            compute(off0, 0)

            @pl.when(off0 + 2 * CH < b_per_w)
            def _():
                fire(off0 + 2 * CH, 0)

            wait(1)
            compute(off0 + CH, 1)

    return sc_kernel(heads, relations, tails, ent3, relp)
